# baseline (speedup 1.0000x reference)
import functools

import jax
import jax.numpy as jnp
from jax import lax
from jax.experimental import pallas as pl
from jax.experimental.pallas import tpu as pltpu
from jax.experimental.pallas import tpu_sc as plsc


N_NODES = 10000
N_MSG = 32
N_FEAT = 128
LANES = 16

NUM_CORES = 2
NUM_SUBCORES = 16
NUM_WORKERS = NUM_CORES * NUM_SUBCORES

C_NODES = 8
N_CHUNKS = N_NODES // C_NODES
ROWS = C_NODES * N_MSG
T_MAX = (N_CHUNKS + NUM_WORKERS - 1) // NUM_WORKERS
N_PAIRS = T_MAX // 2


def _acc_node(buf, j, ob):
    accs = []
    for ch in range(N_FEAT // LANES):
        sl = pl.ds(ch * LANES, LANES)
        acc = buf[j * N_MSG, sl]
        for r in range(1, N_MSG):
            acc = acc + buf[j * N_MSG + r, sl]
        accs.append((sl, acc))
    for sl, acc in accs:
        ob[j, sl] = acc


def _sc_body(x_hbm, o_hbm, b0, b1, ob0, ob1, s0, s1, so0, so1):
    c = lax.axis_index("c")
    s = lax.axis_index("s")
    w = s * NUM_CORES + c
    last = jnp.where(w < N_CHUNKS - (T_MAX - 1) * NUM_WORKERS,
                     w + NUM_WORKERS * (T_MAX - 1),
                     w + NUM_WORKERS * (T_MAX - 2))

    def chunk_idx(t):
        return jnp.minimum(w + NUM_WORKERS * t, last)

    pltpu.async_copy(x_hbm.at[chunk_idx(0)], b0, s0)
    pltpu.async_copy(x_hbm.at[chunk_idx(1)], b1, s1)

    def pair(i, carry):
        for par, (buf, ob, si, so) in enumerate(
                ((b0, ob0, s0, so0), (b1, ob1, s1, so1))):
            t = 2 * i + par
            @pl.when(i > 0)
            def _():
                pltpu.make_async_copy(ob, o_hbm.at[0], so).wait()
            pltpu.make_async_copy(x_hbm.at[0], buf, si).wait()
            def node(j, carry2):
                _acc_node(buf, j, ob)
                return carry2
            lax.fori_loop(0, C_NODES, node, 0, unroll=False)
            @pl.when(t + 2 < T_MAX)
            def _():
                pltpu.async_copy(x_hbm.at[chunk_idx(t + 2)], buf, si)
            pltpu.async_copy(ob, o_hbm.at[chunk_idx(t)], so)
        return carry

    lax.fori_loop(0, N_PAIRS, pair, 0, unroll=False)
    pltpu.make_async_copy(ob0, o_hbm.at[0], so0).wait()
    pltpu.make_async_copy(ob1, o_hbm.at[0], so1).wait()


@functools.partial(jax.jit, static_argnames=())
def _sc_reduce(x):
    mesh = plsc.VectorSubcoreMesh(core_axis_name="c", subcore_axis_name="s")
    f = pl.kernel(
        _sc_body,
        out_type=jax.ShapeDtypeStruct((N_CHUNKS, C_NODES, N_FEAT), jnp.float32),
        mesh=mesh,
        scratch_types=[
            pltpu.VMEM((ROWS, N_FEAT), jnp.float32),
            pltpu.VMEM((ROWS, N_FEAT), jnp.float32),
            pltpu.VMEM((C_NODES, N_FEAT), jnp.float32),
            pltpu.VMEM((C_NODES, N_FEAT), jnp.float32),
            pltpu.SemaphoreType.DMA,
            pltpu.SemaphoreType.DMA,
            pltpu.SemaphoreType.DMA,
            pltpu.SemaphoreType.DMA,
        ],
    )
    return f(x)


def kernel(messages):
    x = messages.reshape(N_CHUNKS, ROWS, N_FEAT)
    out = _sc_reduce(x)
    return out.reshape(1, N_NODES, N_FEAT)


# speedup vs baseline: 1.2936x; 1.2936x over previous
import functools

import jax
import jax.numpy as jnp
from jax import lax
from jax.experimental import pallas as pl
from jax.experimental.pallas import tpu as pltpu
from jax.experimental.pallas import tpu_sc as plsc


N_NODES = 10000
N_MSG = 32
N_FEAT = 128
LANES = 16

NUM_CORES = 2
NUM_SUBCORES = 16
NUM_WORKERS = NUM_CORES * NUM_SUBCORES

C_NODES = 8
ROWS = C_NODES * N_MSG

SC_CHUNKS = 448
SC_NODES = SC_CHUNKS * C_NODES
TC_NODES = N_NODES - SC_NODES
TC_CHUNKS = TC_NODES // C_NODES
TC_BLK = 400

T_PER_W = SC_CHUNKS // NUM_WORKERS
N_PAIRS = T_PER_W // 2


def _acc_node(buf, j, ob):
    accs = []
    for ch in range(N_FEAT // LANES):
        sl = pl.ds(ch * LANES, LANES)
        acc = buf[j * N_MSG, sl]
        for r in range(1, N_MSG):
            acc = acc + buf[j * N_MSG + r, sl]
        accs.append((sl, acc))
    for sl, acc in accs:
        ob[j, sl] = acc


def _sc_body(x_hbm, o_hbm, b0, b1, ob0, ob1, s0, s1, so0, so1):
    c = lax.axis_index("c")
    s = lax.axis_index("s")
    w = s * NUM_CORES + c
    base = TC_CHUNKS

    pltpu.async_copy(x_hbm.at[base + w], b0, s0)
    pltpu.async_copy(x_hbm.at[base + w + NUM_WORKERS], b1, s1)

    def pair(i, carry):
        for par, (buf, ob, si, so) in enumerate(
                ((b0, ob0, s0, so0), (b1, ob1, s1, so1))):
            t = 2 * i + par
            chunk = base + w + NUM_WORKERS * t
            @pl.when(i > 0)
            def _():
                pltpu.make_async_copy(ob, o_hbm.at[0], so).wait()
            pltpu.make_async_copy(x_hbm.at[0], buf, si).wait()
            def node(j, carry2):
                _acc_node(buf, j, ob)
                return carry2
            lax.fori_loop(0, C_NODES, node, 0, unroll=False)
            @pl.when(t + 2 < T_PER_W)
            def _():
                pltpu.async_copy(
                    x_hbm.at[chunk + 2 * NUM_WORKERS], buf, si)
            pltpu.async_copy(ob, o_hbm.at[chunk - base], so)
        return carry

    lax.fori_loop(0, N_PAIRS, pair, 0, unroll=False)
    pltpu.make_async_copy(ob0, o_hbm.at[0], so0).wait()
    pltpu.make_async_copy(ob1, o_hbm.at[0], so1).wait()


def _sc_reduce(x):
    mesh = plsc.VectorSubcoreMesh(core_axis_name="c", subcore_axis_name="s")
    f = pl.kernel(
        _sc_body,
        out_type=jax.ShapeDtypeStruct((SC_CHUNKS, C_NODES, N_FEAT),
                                      jnp.float32),
        mesh=mesh,
        scratch_types=[
            pltpu.VMEM((ROWS, N_FEAT), jnp.float32),
            pltpu.VMEM((ROWS, N_FEAT), jnp.float32),
            pltpu.VMEM((C_NODES, N_FEAT), jnp.float32),
            pltpu.VMEM((C_NODES, N_FEAT), jnp.float32),
            pltpu.SemaphoreType.DMA,
            pltpu.SemaphoreType.DMA,
            pltpu.SemaphoreType.DMA,
            pltpu.SemaphoreType.DMA,
        ],
    )
    return f(x)


def _tc_reduce_body(x_ref, o_ref):
    o_ref[...] = jnp.sum(x_ref[...], axis=1)


def _tc_reduce(x):
    return pl.pallas_call(
        _tc_reduce_body,
        grid=(pl.cdiv(TC_NODES, TC_BLK),),
        in_specs=[pl.BlockSpec((TC_BLK, N_MSG, N_FEAT), lambda i: (i, 0, 0))],
        out_specs=pl.BlockSpec((TC_BLK, N_FEAT), lambda i: (i, 0)),
        out_shape=jax.ShapeDtypeStruct((TC_NODES, N_FEAT), jnp.float32),
    )(x)


def kernel(messages):
    x = messages.reshape(N_NODES, N_MSG, N_FEAT)
    tc_out = _tc_reduce(x)
    sc_out = _sc_reduce(x.reshape(N_NODES // C_NODES, ROWS, N_FEAT))
    out = jnp.concatenate([tc_out, sc_out.reshape(SC_NODES, N_FEAT)], axis=0)
    return out.reshape(1, N_NODES, N_FEAT)


# speedup vs baseline: 1.3192x; 1.0198x over previous
import functools

import jax
import jax.numpy as jnp
from jax import lax
from jax.experimental import pallas as pl
from jax.experimental.pallas import tpu as pltpu
from jax.experimental.pallas import tpu_sc as plsc


N_NODES = 10000
N_MSG = 32
N_FEAT = 128
LANES = 16

NUM_CORES = 2
NUM_SUBCORES = 16
NUM_WORKERS = NUM_CORES * NUM_SUBCORES

C_NODES = 8
ROWS = C_NODES * N_MSG

SC_CHUNKS = 192
SC_NODES = SC_CHUNKS * C_NODES
TC_NODES = N_NODES - SC_NODES
TC_CHUNKS = TC_NODES // C_NODES
TC_BLK = 400

T_PER_W = SC_CHUNKS // NUM_WORKERS
N_PAIRS = T_PER_W // 2


def _acc_node(buf, j, ob):
    accs = []
    for ch in range(N_FEAT // LANES):
        sl = pl.ds(ch * LANES, LANES)
        acc = buf[j * N_MSG, sl]
        for r in range(1, N_MSG):
            acc = acc + buf[j * N_MSG + r, sl]
        accs.append((sl, acc))
    for sl, acc in accs:
        ob[j, sl] = acc


def _sc_body(x_hbm, o_hbm, b0, b1, ob0, ob1, s0, s1, so0, so1):
    c = lax.axis_index("c")
    s = lax.axis_index("s")
    w = s * NUM_CORES + c
    base = TC_CHUNKS

    pltpu.async_copy(x_hbm.at[base + w], b0, s0)
    pltpu.async_copy(x_hbm.at[base + w + NUM_WORKERS], b1, s1)

    def pair(i, carry):
        for par, (buf, ob, si, so) in enumerate(
                ((b0, ob0, s0, so0), (b1, ob1, s1, so1))):
            t = 2 * i + par
            chunk = base + w + NUM_WORKERS * t
            @pl.when(i > 0)
            def _():
                pltpu.make_async_copy(ob, o_hbm.at[0], so).wait()
            pltpu.make_async_copy(x_hbm.at[0], buf, si).wait()
            def node(j, carry2):
                _acc_node(buf, j, ob)
                return carry2
            lax.fori_loop(0, C_NODES, node, 0, unroll=False)
            @pl.when(t + 2 < T_PER_W)
            def _():
                pltpu.async_copy(
                    x_hbm.at[chunk + 2 * NUM_WORKERS], buf, si)
            pltpu.async_copy(ob, o_hbm.at[chunk - base], so)
        return carry

    lax.fori_loop(0, N_PAIRS, pair, 0, unroll=False)
    pltpu.make_async_copy(ob0, o_hbm.at[0], so0).wait()
    pltpu.make_async_copy(ob1, o_hbm.at[0], so1).wait()


def _sc_reduce(x):
    mesh = plsc.VectorSubcoreMesh(core_axis_name="c", subcore_axis_name="s")
    f = pl.kernel(
        _sc_body,
        out_type=jax.ShapeDtypeStruct((SC_CHUNKS, C_NODES, N_FEAT),
                                      jnp.float32),
        mesh=mesh,
        scratch_types=[
            pltpu.VMEM((ROWS, N_FEAT), jnp.float32),
            pltpu.VMEM((ROWS, N_FEAT), jnp.float32),
            pltpu.VMEM((C_NODES, N_FEAT), jnp.float32),
            pltpu.VMEM((C_NODES, N_FEAT), jnp.float32),
            pltpu.SemaphoreType.DMA,
            pltpu.SemaphoreType.DMA,
            pltpu.SemaphoreType.DMA,
            pltpu.SemaphoreType.DMA,
        ],
    )
    return f(x)


def _tc_reduce_body(x_ref, o_ref):
    o_ref[...] = jnp.sum(x_ref[...], axis=1)


def _tc_reduce(x):
    return pl.pallas_call(
        _tc_reduce_body,
        grid=(pl.cdiv(TC_NODES, TC_BLK),),
        in_specs=[pl.BlockSpec((TC_BLK, N_MSG, N_FEAT), lambda i: (i, 0, 0))],
        out_specs=pl.BlockSpec((TC_BLK, N_FEAT), lambda i: (i, 0)),
        out_shape=jax.ShapeDtypeStruct((TC_NODES, N_FEAT), jnp.float32),
    )(x)


def kernel(messages):
    x = messages.reshape(N_NODES, N_MSG, N_FEAT)
    tc_out = _tc_reduce(x)
    sc_out = _sc_reduce(x.reshape(N_NODES // C_NODES, ROWS, N_FEAT))
    out = jnp.concatenate([tc_out, sc_out.reshape(SC_NODES, N_FEAT)], axis=0)
    return out.reshape(1, N_NODES, N_FEAT)


# speedup vs baseline: 1.3488x; 1.0225x over previous
import functools

import jax
import jax.numpy as jnp
from jax import lax
from jax.experimental import pallas as pl
from jax.experimental.pallas import tpu as pltpu
from jax.experimental.pallas import tpu_sc as plsc


N_NODES = 10000
N_MSG = 32
N_FEAT = 128
LANES = 16

NUM_CORES = 2
NUM_SUBCORES = 16
NUM_WORKERS = NUM_CORES * NUM_SUBCORES

C_NODES = 8
ROWS = C_NODES * N_MSG

SC_CHUNKS = 64
SC_NODES = SC_CHUNKS * C_NODES
TC_NODES = N_NODES - SC_NODES
TC_CHUNKS = TC_NODES // C_NODES
TC_BLK = 400

T_PER_W = SC_CHUNKS // NUM_WORKERS
N_PAIRS = T_PER_W // 2


def _acc_node(buf, j, ob):
    accs = []
    for ch in range(N_FEAT // LANES):
        sl = pl.ds(ch * LANES, LANES)
        acc = buf[j * N_MSG, sl]
        for r in range(1, N_MSG):
            acc = acc + buf[j * N_MSG + r, sl]
        accs.append((sl, acc))
    for sl, acc in accs:
        ob[j, sl] = acc


def _sc_body(x_hbm, o_hbm, b0, b1, ob0, ob1, s0, s1, so0, so1):
    c = lax.axis_index("c")
    s = lax.axis_index("s")
    w = s * NUM_CORES + c
    base = TC_CHUNKS

    pltpu.async_copy(x_hbm.at[base + w], b0, s0)
    pltpu.async_copy(x_hbm.at[base + w + NUM_WORKERS], b1, s1)

    def pair(i, carry):
        for par, (buf, ob, si, so) in enumerate(
                ((b0, ob0, s0, so0), (b1, ob1, s1, so1))):
            t = 2 * i + par
            chunk = base + w + NUM_WORKERS * t
            @pl.when(i > 0)
            def _():
                pltpu.make_async_copy(ob, o_hbm.at[0], so).wait()
            pltpu.make_async_copy(x_hbm.at[0], buf, si).wait()
            def node(j, carry2):
                _acc_node(buf, j, ob)
                return carry2
            lax.fori_loop(0, C_NODES, node, 0, unroll=False)
            @pl.when(t + 2 < T_PER_W)
            def _():
                pltpu.async_copy(
                    x_hbm.at[chunk + 2 * NUM_WORKERS], buf, si)
            pltpu.async_copy(ob, o_hbm.at[chunk - base], so)
        return carry

    lax.fori_loop(0, N_PAIRS, pair, 0, unroll=False)
    pltpu.make_async_copy(ob0, o_hbm.at[0], so0).wait()
    pltpu.make_async_copy(ob1, o_hbm.at[0], so1).wait()


def _sc_reduce(x):
    mesh = plsc.VectorSubcoreMesh(core_axis_name="c", subcore_axis_name="s")
    f = pl.kernel(
        _sc_body,
        out_type=jax.ShapeDtypeStruct((SC_CHUNKS, C_NODES, N_FEAT),
                                      jnp.float32),
        mesh=mesh,
        scratch_types=[
            pltpu.VMEM((ROWS, N_FEAT), jnp.float32),
            pltpu.VMEM((ROWS, N_FEAT), jnp.float32),
            pltpu.VMEM((C_NODES, N_FEAT), jnp.float32),
            pltpu.VMEM((C_NODES, N_FEAT), jnp.float32),
            pltpu.SemaphoreType.DMA,
            pltpu.SemaphoreType.DMA,
            pltpu.SemaphoreType.DMA,
            pltpu.SemaphoreType.DMA,
        ],
    )
    return f(x)


def _tc_reduce_body(x_ref, o_ref):
    o_ref[...] = jnp.sum(x_ref[...], axis=1)


def _tc_reduce(x):
    return pl.pallas_call(
        _tc_reduce_body,
        grid=(pl.cdiv(TC_NODES, TC_BLK),),
        in_specs=[pl.BlockSpec((TC_BLK, N_MSG, N_FEAT), lambda i: (i, 0, 0))],
        out_specs=pl.BlockSpec((TC_BLK, N_FEAT), lambda i: (i, 0)),
        out_shape=jax.ShapeDtypeStruct((TC_NODES, N_FEAT), jnp.float32),
    )(x)


def kernel(messages):
    x = messages.reshape(N_NODES, N_MSG, N_FEAT)
    tc_out = _tc_reduce(x)
    sc_out = _sc_reduce(x.reshape(N_NODES // C_NODES, ROWS, N_FEAT))
    out = jnp.concatenate([tc_out, sc_out.reshape(SC_NODES, N_FEAT)], axis=0)
    return out.reshape(1, N_NODES, N_FEAT)
